# baseline (speedup 1.0000x reference)
import functools

import jax
import jax.numpy as jnp
from jax import lax
from jax.experimental import pallas as pl
from jax.experimental.pallas import tpu as pltpu
from jax.experimental.pallas import tpu_sc as plsc

N = 10000
E = 160000
G = 8
NUM_POINTS = 1024

NC = 2
NS = 16
NW = NC * NS

CHUNK_G = 128
CPW = 40
E_PAD = NW * CPW * CHUNK_G
BE = 512
NBLK = E_PAD // BE
CPW2 = 3
NIDX2 = NW * CPW2 * CHUNK_G


def _mesh():
    return plsc.VectorSubcoreMesh(core_axis_name="c", subcore_axis_name="s",
                                  num_cores=NC, num_subcores=NS)


def _wid():
    return lax.axis_index("s") * NC + lax.axis_index("c")


def _gather2_body(dst_hbm, src_hbm, p_hbm, q_hbm, pd_hbm, qs_hbm,
                  idx_d, idx_s, rows_p, rows_q, sem):
    wid = _wid()

    def chunk(i, carry):
        base = (wid * CPW + i) * CHUNK_G
        pltpu.sync_copy(dst_hbm.at[pl.ds(base, CHUNK_G)], idx_d)
        pltpu.sync_copy(src_hbm.at[pl.ds(base, CHUNK_G)], idx_s)
        pltpu.async_copy(p_hbm.at[idx_d], rows_p, sem).wait()
        pltpu.async_copy(q_hbm.at[idx_s], rows_q, sem).wait()
        pltpu.sync_copy(rows_p, pd_hbm.at[pl.ds(base, CHUNK_G)])
        pltpu.sync_copy(rows_q, qs_hbm.at[pl.ds(base, CHUNK_G)])
        return carry

    lax.fori_loop(0, CPW, chunk, 0)


def _make_gather2(h):
    out = jax.ShapeDtypeStruct((E_PAD, h), jnp.float32)
    return pl.kernel(
        _gather2_body,
        out_type=[out, out],
        mesh=_mesh(),
        scratch_types=[
            pltpu.VMEM((CHUNK_G,), jnp.int32),
            pltpu.VMEM((CHUNK_G,), jnp.int32),
            pltpu.VMEM((CHUNK_G, h), jnp.float32),
            pltpu.VMEM((CHUNK_G, h), jnp.float32),
            pltpu.SemaphoreType.DMA,
        ],
    )


def _gather1_body(idx_hbm, s_hbm, out_hbm, idxv, rows, sem):
    wid = _wid()

    def chunk(i, carry):
        base = (wid * CPW2 + i) * CHUNK_G
        pltpu.sync_copy(idx_hbm.at[pl.ds(base, CHUNK_G)], idxv)
        pltpu.async_copy(s_hbm.at[idxv], rows, sem).wait()
        pltpu.sync_copy(rows, out_hbm.at[pl.ds(base, CHUNK_G)])
        return carry

    lax.fori_loop(0, CPW2, chunk, 0)


def _make_gather1(h):
    return pl.kernel(
        _gather1_body,
        out_type=jax.ShapeDtypeStruct((NIDX2, h), jnp.float32),
        mesh=_mesh(),
        scratch_types=[
            pltpu.VMEM((CHUNK_G,), jnp.int32),
            pltpu.VMEM((CHUNK_G, h), jnp.float32),
            pltpu.SemaphoreType.DMA,
        ],
    )


def _pq_body(hin, x_ref, wa_ref, ba_ref, p_ref, q_ref):
    xb = x_ref[...]
    wa = wa_ref[...]
    wq = wa[hin:, :]
    wp = wa[:hin, :] - wq
    p_ref[...] = (jnp.dot(xb, wp, preferred_element_type=jnp.float32)
                  + ba_ref[...])
    q_ref[...] = jnp.dot(xb, wq, preferred_element_type=jnp.float32)


def _pq_call(x, wa, ba):
    n, hin = x.shape
    h = wa.shape[1]
    bn = 1000
    out = jax.ShapeDtypeStruct((n, h), jnp.float32)
    return pl.pallas_call(
        functools.partial(_pq_body, hin),
        grid=(n // bn,),
        in_specs=[
            pl.BlockSpec((bn, hin), lambda i: (i, 0)),
            pl.BlockSpec((2 * hin, h), lambda i: (0, 0)),
            pl.BlockSpec((1, h), lambda i: (0, 0)),
        ],
        out_specs=[
            pl.BlockSpec((bn, h), lambda i: (i, 0)),
            pl.BlockSpec((bn, h), lambda i: (i, 0)),
        ],
        out_shape=[out, out],
    )(x, wa, ba.reshape(1, h))


def _msg_scan_body(pd_ref, qs_ref, d_ref, wb_ref, bb_ref, s_ref):
    hact = jnp.maximum(pd_ref[...] + qs_ref[...], 0.0)
    m = (jnp.dot(hact, wb_ref[...], preferred_element_type=jnp.float32)
         + bb_ref[...])
    dcol = d_ref[...]
    k = 1
    while k < BE:
        dsh = jnp.concatenate(
            [dcol[k:], jnp.full((k, 1), -2, jnp.int32)], axis=0)
        msh = jnp.concatenate(
            [m[k:], jnp.zeros((k, m.shape[1]), m.dtype)], axis=0)
        m = jnp.where(dsh == dcol, jnp.maximum(m, msh), m)
        k *= 2
    s_ref[...] = m


def _msg_scan_call(pd, qs, dcol, wb, bb):
    e, h = pd.shape
    return pl.pallas_call(
        _msg_scan_body,
        grid=(e // BE,),
        in_specs=[
            pl.BlockSpec((BE, h), lambda i: (i, 0)),
            pl.BlockSpec((BE, h), lambda i: (i, 0)),
            pl.BlockSpec((BE, 1), lambda i: (i, 0)),
            pl.BlockSpec((h, h), lambda i: (0, 0)),
            pl.BlockSpec((1, h), lambda i: (0, 0)),
        ],
        out_specs=pl.BlockSpec((BE, h), lambda i: (i, 0)),
        out_shape=jax.ShapeDtypeStruct((e, h), jnp.float32),
    )(pd, qs, dcol, wb, bb.reshape(1, h))


def _merge_body(g_ref, cnt_ref, sb_ref, bdst_ref, o_ref):
    o_ref[...] = jnp.where(cnt_ref[...] > 0, g_ref[...], 0.0)

    def mrg(i, carry):
        n = bdst_ref[i]

        @pl.when(n >= 0)
        def _():
            cur = o_ref[pl.ds(n, 1), :]
            o_ref[pl.ds(n, 1), :] = jnp.maximum(cur, sb_ref[pl.ds(i, 1), :])

        return carry

    lax.fori_loop(0, NBLK, mrg, 0)


def _merge_call(g, cnt, sb, bdst):
    n, h = g.shape
    return pl.pallas_call(
        _merge_body,
        in_specs=[
            pl.BlockSpec((n, h), lambda: (0, 0)),
            pl.BlockSpec((n, 1), lambda: (0, 0)),
            pl.BlockSpec((NBLK, h), lambda: (0, 0)),
            pl.BlockSpec(memory_space=pltpu.SMEM),
        ],
        out_specs=pl.BlockSpec((n, h), lambda: (0, 0)),
        out_shape=jax.ShapeDtypeStruct((n, h), jnp.float32),
    )(g, cnt, sb, bdst)


def _dec_body(x3_ref, b_ref, wfc_ref, bfc_ref, wd1_ref, bd1_ref,
              wd2_ref, bd2_ref, o_ref):
    x3 = x3_ref[...]
    bcol = b_ref[...]
    rows = []
    for g in range(G):
        mg = jnp.where(bcol == g, x3, -jnp.inf)
        rows.append(jnp.max(mg, axis=0, keepdims=True))
    pooled = jnp.concatenate(rows, axis=0)
    pooled = jnp.where(pooled > jnp.float32(-3e38), pooled, 0.0)
    z = (jnp.dot(pooled, wfc_ref[...], preferred_element_type=jnp.float32)
         + bfc_ref[...])
    hh = jnp.maximum(
        jnp.dot(z, wd1_ref[...], preferred_element_type=jnp.float32)
        + bd1_ref[...], 0.0)
    o_ref[...] = (jnp.dot(hh, wd2_ref[...], preferred_element_type=jnp.float32)
                  + bd2_ref[...])


def _dec_call(x3, batch, wfc, bfc, wd1, bd1, wd2, bd2):
    n, h = x3.shape
    dout = wd2.shape[1]
    return pl.pallas_call(
        _dec_body,
        in_specs=[pl.BlockSpec((n, h), lambda: (0, 0)),
                  pl.BlockSpec((n, 1), lambda: (0, 0)),
                  pl.BlockSpec(wfc.shape, lambda: (0, 0)),
                  pl.BlockSpec((1, wfc.shape[1]), lambda: (0, 0)),
                  pl.BlockSpec(wd1.shape, lambda: (0, 0)),
                  pl.BlockSpec((1, wd1.shape[1]), lambda: (0, 0)),
                  pl.BlockSpec(wd2.shape, lambda: (0, 0)),
                  pl.BlockSpec((1, dout), lambda: (0, 0))],
        out_specs=pl.BlockSpec((G, dout), lambda: (0, 0)),
        out_shape=jax.ShapeDtypeStruct((G, dout), jnp.float32),
    )(x3, batch.reshape(n, 1), wfc, bfc.reshape(1, -1),
      wd1, bd1.reshape(1, -1), wd2, bd2.reshape(1, -1))


def kernel(x, edge_index, batch, W1a, b1a, W1b, b1b, W2a, b2a, W2b, b2b,
           W3a, b3a, W3b, b3b, Wfc, bfc, Wd1, bd1, Wd2, bd2):
    src = edge_index[0]
    dst = edge_index[1]

    order = jnp.argsort(dst)
    dst_s = dst[order]
    src_s = src[order]
    dstp0 = jnp.pad(dst_s, (0, E_PAD - E))
    srcp0 = jnp.pad(src_s, (0, E_PAD - E))
    dstm1 = jnp.pad(dst_s, (0, E_PAD - E),
                    constant_values=-1).reshape(E_PAD, 1)
    bounds = jnp.searchsorted(
        dst_s, jnp.arange(N + 1, dtype=jnp.int32)).astype(jnp.int32)
    start = jnp.minimum(bounds[:N], E - 1)
    cnt = (bounds[1:] - bounds[:N]).reshape(N, 1)
    blockstarts = jnp.arange(0, E_PAD, BE, dtype=jnp.int32)
    bdst = dstm1[::BE, 0]
    idx2 = jnp.pad(jnp.concatenate([start, blockstarts]),
                   (0, NIDX2 - (N + NBLK)))

    w1a_e = jnp.pad(W1a, ((0, 0), (0, 64)))
    b1a_e = jnp.pad(b1a, (0, 64))
    w1b_e = jnp.pad(W1b, ((0, 64), (0, 64)))
    b1b_e = jnp.pad(b1b, (0, 64))
    w2a_e = jnp.concatenate([jnp.pad(W2a[:64], ((0, 64), (0, 0))),
                             jnp.pad(W2a[64:], ((0, 64), (0, 0)))], axis=0)

    xl = x
    for wa, ba, wb, bb in ((w1a_e, b1a_e, w1b_e, b1b_e),
                           (w2a_e, b2a, W2b, b2b),
                           (W3a, b3a, W3b, b3b)):
        h = wa.shape[1]
        p, q = _pq_call(xl, wa, ba)
        pd, qs = _make_gather2(h)(dstp0, srcp0, p, q)
        s = _msg_scan_call(pd, qs, dstm1, wb, bb)
        sg = _make_gather1(h)(idx2, s)
        xl = _merge_call(sg[:N], cnt, sg[N:N + NBLK], bdst)

    out = _dec_call(xl, batch, Wfc, bfc, Wd1, bd1, Wd2, bd2)
    return out.reshape(G, NUM_POINTS, 3)
